# baseline (speedup 1.0000x reference)
import functools

import jax
import jax.numpy as jnp
from jax import lax
from jax.experimental import pallas as pl
from jax.experimental.pallas import tpu as pltpu
from jax.experimental.pallas import tpu_sc as plsc

INPUT_DIM = 2048
SIDE_DIM = 2048
CONTEXT_DIM = 4
NUM_CTX = 2 ** CONTEXT_DIM
BATCH = 4096
BB = 256

NC = 2
NS = 16
NW = NC * NS
BPW = BATCH // NW
LANES = 16
ROW = NUM_CTX * BPW + BPW


def _tc_body(side_ref, logit_ref, v_ref, b_ref, w_ref, bc_ref, buf_ref):
    proj = jnp.dot(v_ref[...], side_ref[...],
                   preferred_element_type=jnp.float32)
    bits = (proj > b_ref[...]).astype(jnp.float32)
    ctxf = jnp.sum(bits * bc_ref[...], axis=0)
    dots = jnp.dot(w_ref[...], logit_ref[...],
                   preferred_element_type=jnp.float32)
    wpb = BB // BPW
    merged = jnp.concatenate(
        [dots.reshape(NUM_CTX, wpb, BPW).swapaxes(0, 1).reshape(wpb, NUM_CTX * BPW),
         ctxf.reshape(wpb, BPW)], axis=1)
    buf_ref[...] = merged.reshape(wpb, 1, ROW)


def _sc_route(buf_hbm, out_hbm, buf_v, out_v):
    wid = lax.axis_index("s") * NC + lax.axis_index("c")
    base = wid * BPW
    pltpu.sync_copy(buf_hbm.at[wid, 0], buf_v)
    for i in range(BPW // LANES):
        rows = buf_v[pl.ds(NUM_CTX * BPW + i * LANES, LANES)].astype(jnp.int32)
        acc = jnp.zeros((LANES,), jnp.float32)
        for k in range(NUM_CTX):
            val = buf_v[pl.ds(k * BPW + i * LANES, LANES)]
            acc = jnp.where(rows == k, val, acc)
        out_v[pl.ds(i * LANES, LANES)] = acc
    pltpu.sync_copy(out_v, out_hbm.at[pl.ds(base, BPW)])


def kernel(logit_previous, side_information, v, b, weights, boolean_converter):
    grid = BATCH // BB
    buf = pl.pallas_call(
        _tc_body,
        grid=(grid,),
        in_specs=[
            pl.BlockSpec((SIDE_DIM, BB), lambda i: (0, i)),
            pl.BlockSpec((INPUT_DIM, BB), lambda i: (0, i)),
            pl.BlockSpec((CONTEXT_DIM, SIDE_DIM), lambda i: (0, 0)),
            pl.BlockSpec((CONTEXT_DIM, 1), lambda i: (0, 0)),
            pl.BlockSpec((NUM_CTX, INPUT_DIM), lambda i: (0, 0)),
            pl.BlockSpec((CONTEXT_DIM, 1), lambda i: (0, 0)),
        ],
        out_specs=pl.BlockSpec((BB // BPW, 1, ROW), lambda i: (i, 0, 0)),
        out_shape=jax.ShapeDtypeStruct((NW, 1, ROW), jnp.float32),
    )(side_information, logit_previous, v, b, weights, boolean_converter)

    route = functools.partial(
        pl.kernel,
        mesh=plsc.VectorSubcoreMesh(core_axis_name="c", subcore_axis_name="s"),
        out_type=jax.ShapeDtypeStruct((BATCH,), jnp.float32),
        scratch_types=[
            pltpu.VMEM((ROW,), jnp.float32),
            pltpu.VMEM((BPW,), jnp.float32),
        ],
    )(_sc_route)
    return route(buf)


# speedup vs baseline: 1.0258x; 1.0258x over previous
import functools

import jax
import jax.numpy as jnp
from jax import lax
from jax.experimental import pallas as pl
from jax.experimental.pallas import tpu as pltpu
from jax.experimental.pallas import tpu_sc as plsc

INPUT_DIM = 2048
SIDE_DIM = 2048
CONTEXT_DIM = 4
NUM_CTX = 2 ** CONTEXT_DIM
BATCH = 4096
BB = 512
HALF = BATCH // 2

NC = 2
NS = 16
NW = NC * NS
BPW = 128
NWH = HALF // BPW
LANES = 16
ROW = NUM_CTX * BPW + BPW


def _gates_and_dots(side_ref, logit_ref, v_ref, b_ref, w_ref, bc_ref):
    proj = jnp.dot(v_ref[...], side_ref[...],
                   preferred_element_type=jnp.float32)
    bits = (proj > b_ref[...]).astype(jnp.float32)
    ctxf = jnp.sum(bits * bc_ref[...], axis=0)
    dots = jnp.dot(w_ref[...], logit_ref[...],
                   preferred_element_type=jnp.float32)
    return ctxf, dots


def _tc_stage_body(side_ref, logit_ref, v_ref, b_ref, w_ref, bc_ref, buf_ref):
    ctxf, dots = _gates_and_dots(side_ref, logit_ref, v_ref, b_ref, w_ref, bc_ref)
    wpb = BB // BPW
    merged = jnp.concatenate(
        [dots.reshape(NUM_CTX, wpb, BPW).swapaxes(0, 1).reshape(wpb, NUM_CTX * BPW),
         ctxf.reshape(wpb, BPW)], axis=1)
    buf_ref[...] = merged.reshape(wpb, 1, ROW)


def _tc_select_body(side_ref, logit_ref, v_ref, b_ref, w_ref, bc_ref, out_ref):
    ctxf, dots = _gates_and_dots(side_ref, logit_ref, v_ref, b_ref, w_ref, bc_ref)
    ctx = ctxf.astype(jnp.int32)
    row_ids = lax.broadcasted_iota(jnp.int32, (NUM_CTX, dots.shape[1]), 0)
    onehot = (row_ids == ctx[None, :]).astype(jnp.float32)
    out_ref[0, :] = jnp.sum(onehot * dots, axis=0)


def _sc_route(buf_hbm, out_hbm, buf_v, out_v):
    wid = lax.axis_index("s") * NC + lax.axis_index("c")

    @pl.when(wid < NWH)
    def _():
        base = wid * BPW
        pltpu.sync_copy(buf_hbm.at[wid, 0], buf_v)
        for i in range(BPW // LANES):
            rows = buf_v[pl.ds(NUM_CTX * BPW + i * LANES, LANES)].astype(jnp.int32)
            acc = jnp.zeros((LANES,), jnp.float32)
            for k in range(NUM_CTX):
                val = buf_v[pl.ds(k * BPW + i * LANES, LANES)]
                acc = jnp.where(rows == k, val, acc)
            out_v[pl.ds(i * LANES, LANES)] = acc
        pltpu.sync_copy(out_v, out_hbm.at[pl.ds(base, BPW)])


_COMMON_IN_SPECS = [
    pl.BlockSpec((CONTEXT_DIM, SIDE_DIM), lambda i: (0, 0)),
    pl.BlockSpec((CONTEXT_DIM, 1), lambda i: (0, 0)),
    pl.BlockSpec((NUM_CTX, INPUT_DIM), lambda i: (0, 0)),
    pl.BlockSpec((CONTEXT_DIM, 1), lambda i: (0, 0)),
]


def kernel(logit_previous, side_information, v, b, weights, boolean_converter):
    half_grid = HALF // BB

    buf = pl.pallas_call(
        _tc_stage_body,
        grid=(half_grid,),
        in_specs=[
            pl.BlockSpec((SIDE_DIM, BB), lambda i: (0, i)),
            pl.BlockSpec((INPUT_DIM, BB), lambda i: (0, i)),
        ] + _COMMON_IN_SPECS,
        out_specs=pl.BlockSpec((BB // BPW, 1, ROW), lambda i: (i, 0, 0)),
        out_shape=jax.ShapeDtypeStruct((NWH, 1, ROW), jnp.float32),
    )(side_information, logit_previous, v, b, weights, boolean_converter)

    route = functools.partial(
        pl.kernel,
        mesh=plsc.VectorSubcoreMesh(core_axis_name="c", subcore_axis_name="s"),
        out_type=jax.ShapeDtypeStruct((HALF,), jnp.float32),
        scratch_types=[
            pltpu.VMEM((ROW,), jnp.float32),
            pltpu.VMEM((BPW,), jnp.float32),
        ],
    )(_sc_route)
    out1 = route(buf)

    out2 = pl.pallas_call(
        _tc_select_body,
        grid=(half_grid,),
        in_specs=[
            pl.BlockSpec((SIDE_DIM, BB), lambda i: (0, i + half_grid)),
            pl.BlockSpec((INPUT_DIM, BB), lambda i: (0, i + half_grid)),
        ] + _COMMON_IN_SPECS,
        out_specs=pl.BlockSpec((1, BB), lambda i: (0, i)),
        out_shape=jax.ShapeDtypeStruct((1, HALF), jnp.float32),
    )(side_information, logit_previous, v, b, weights, boolean_converter)

    return jnp.concatenate([out1, out2.reshape(HALF)])


# speedup vs baseline: 1.0792x; 1.0520x over previous
import functools

import jax
import jax.numpy as jnp
from jax import lax
from jax.experimental import pallas as pl
from jax.experimental.pallas import tpu as pltpu
from jax.experimental.pallas import tpu_sc as plsc

INPUT_DIM = 2048
SIDE_DIM = 2048
CONTEXT_DIM = 4
NUM_CTX = 2 ** CONTEXT_DIM
BATCH = 4096
BB = 512

NC = 1
NS = 16
NW = NC * NS
BPW = BATCH // NW
LANES = 16
ROW = NUM_CTX * BPW + BPW


def _tc_body(side_ref, logit_ref, v_ref, b_ref, w_ref, bc_ref, buf_ref):
    proj = jnp.dot(v_ref[...], side_ref[...],
                   preferred_element_type=jnp.float32)
    bits = (proj > b_ref[...]).astype(jnp.float32)
    ctxf = jnp.sum(bits * bc_ref[...], axis=0)
    dots = jnp.dot(w_ref[...], logit_ref[...],
                   preferred_element_type=jnp.float32)
    wpb = BB // BPW
    merged = jnp.concatenate(
        [dots.reshape(NUM_CTX, wpb, BPW).swapaxes(0, 1).reshape(wpb, NUM_CTX * BPW),
         ctxf.reshape(wpb, BPW)], axis=1)
    buf_ref[...] = merged.reshape(wpb, 1, ROW)


def _sc_route(buf_hbm, out_hbm, buf_v, out_v):
    wid = lax.axis_index("s") * NC + lax.axis_index("c")
    base = wid * BPW
    pltpu.sync_copy(buf_hbm.at[wid, 0], buf_v)
    for i in range(BPW // LANES):
        rows = buf_v[pl.ds(NUM_CTX * BPW + i * LANES, LANES)].astype(jnp.int32)
        acc = jnp.zeros((LANES,), jnp.float32)
        for k in range(NUM_CTX):
            val = buf_v[pl.ds(k * BPW + i * LANES, LANES)]
            acc = jnp.where(rows == k, val, acc)
        out_v[pl.ds(i * LANES, LANES)] = acc
    pltpu.sync_copy(out_v, out_hbm.at[pl.ds(base, BPW)])


def kernel(logit_previous, side_information, v, b, weights, boolean_converter):
    grid = BATCH // BB
    buf = pl.pallas_call(
        _tc_body,
        grid=(grid,),
        in_specs=[
            pl.BlockSpec((SIDE_DIM, BB), lambda i: (0, i)),
            pl.BlockSpec((INPUT_DIM, BB), lambda i: (0, i)),
            pl.BlockSpec((CONTEXT_DIM, SIDE_DIM), lambda i: (0, 0)),
            pl.BlockSpec((CONTEXT_DIM, 1), lambda i: (0, 0)),
            pl.BlockSpec((NUM_CTX, INPUT_DIM), lambda i: (0, 0)),
            pl.BlockSpec((CONTEXT_DIM, 1), lambda i: (0, 0)),
        ],
        out_specs=pl.BlockSpec((BB // BPW, 1, ROW), lambda i: (i, 0, 0)),
        out_shape=jax.ShapeDtypeStruct((NW, 1, ROW), jnp.float32),
    )(side_information, logit_previous, v, b, weights, boolean_converter)

    route = functools.partial(
        pl.kernel,
        mesh=plsc.VectorSubcoreMesh(core_axis_name="c", subcore_axis_name="s",
                                    num_cores=NC),
        out_type=jax.ShapeDtypeStruct((BATCH,), jnp.float32),
        scratch_types=[
            pltpu.VMEM((ROW,), jnp.float32),
            pltpu.VMEM((BPW,), jnp.float32),
        ],
    )(_sc_route)
    return route(buf)
